# Initial kernel scaffold; baseline (speedup 1.0000x reference)
import functools

import jax
import jax.numpy as jnp
from jax import lax
from jax.experimental import pallas as pl
from jax.experimental.pallas import tpu as pltpu
from jax.experimental.pallas import tpu_sc as plsc

VOCAB = 1000000
EMB = 32
B = 4096
L = 200
N = B * L
NC = 2
NS = 16
NW = NC * NS
PER_W = N // NW
CHUNK = 1024
NCH = PER_W // CHUNK

_mesh = plsc.VectorSubcoreMesh(core_axis_name="c", subcore_axis_name="s")


@functools.partial(
    pl.kernel,
    mesh=_mesh,
    out_type=jax.ShapeDtypeStruct((N, EMB), jnp.float32),
    scratch_types=[
        pltpu.VMEM((CHUNK,), jnp.int32),
        pltpu.VMEM((CHUNK, EMB), jnp.float32),
        pltpu.SemaphoreType.DMA,
    ],
)
def _gather_kernel(idx_hbm, table_hbm, out_hbm, idx_v, rows_v, sem):
    wid = lax.axis_index("s") * NC + lax.axis_index("c")
    base = wid * PER_W

    def body(c, carry):
        off = base + c * CHUNK
        pltpu.sync_copy(idx_hbm.at[pl.ds(off, CHUNK)], idx_v)
        pltpu.async_copy(table_hbm.at[idx_v], rows_v, sem).wait()
        pltpu.sync_copy(rows_v, out_hbm.at[pl.ds(off, CHUNK)])
        return carry

    lax.fori_loop(0, NCH, body, 0)


def kernel(sent_words, embed_weight):
    idx = sent_words.reshape(-1).astype(jnp.int32)
    out = _gather_kernel(idx, embed_weight)
    return out.reshape(B, L, EMB)


# speedup vs baseline: 1.4575x; 1.4575x over previous
import functools

import jax
import jax.numpy as jnp
from jax import lax
from jax.experimental import pallas as pl
from jax.experimental.pallas import tpu as pltpu
from jax.experimental.pallas import tpu_sc as plsc

VOCAB = 1000000
EMB = 32
B = 4096
L = 200
N = B * L
NC = 2
NS = 16
NW = NC * NS
PER_W = N // NW
CHUNK = 1024
NCH = PER_W // CHUNK

_mesh = plsc.VectorSubcoreMesh(core_axis_name="c", subcore_axis_name="s")


@functools.partial(
    pl.kernel,
    mesh=_mesh,
    out_type=jax.ShapeDtypeStruct((N, EMB), jnp.float32),
    compiler_params=pltpu.CompilerParams(use_tc_tiling_on_sc=False),
    scratch_types=[
        pltpu.VMEM((CHUNK,), jnp.int32),
        pltpu.VMEM((CHUNK, EMB), jnp.float32),
        pltpu.SemaphoreType.DMA,
    ],
)
def _gather_kernel(idx_hbm, table_hbm, out_hbm, idx_v, rows_v, sem):
    wid = lax.axis_index("s") * NC + lax.axis_index("c")
    base = wid * PER_W

    def body(c, carry):
        off = base + c * CHUNK
        pltpu.sync_copy(idx_hbm.at[pl.ds(off, CHUNK)], idx_v)
        pltpu.async_copy(table_hbm.at[idx_v], rows_v, sem).wait()
        pltpu.sync_copy(rows_v, out_hbm.at[pl.ds(off, CHUNK)])
        return carry

    lax.fori_loop(0, NCH, body, 0)


def kernel(sent_words, embed_weight):
    idx = sent_words.reshape(-1).astype(jnp.int32)
    out = _gather_kernel(idx, embed_weight)
    return out.reshape(B, L, EMB)


# speedup vs baseline: 1.4943x; 1.0252x over previous
import functools

import jax
import jax.numpy as jnp
from jax import lax
from jax.experimental import pallas as pl
from jax.experimental.pallas import tpu as pltpu
from jax.experimental.pallas import tpu_sc as plsc

VOCAB = 1000000
EMB = 32
B = 4096
L = 200
N = B * L
NC = 2
NS = 16
NW = NC * NS
PER_W = N // NW
CHUNK = 640
NBUF = 4
ROUNDS = PER_W // (CHUNK * NBUF)

_mesh = plsc.VectorSubcoreMesh(core_axis_name="c", subcore_axis_name="s")


@functools.partial(
    pl.kernel,
    mesh=_mesh,
    out_type=jax.ShapeDtypeStruct((N, EMB), jnp.float32),
    compiler_params=pltpu.CompilerParams(use_tc_tiling_on_sc=False),
    scratch_types=[
        pltpu.VMEM((PER_W,), jnp.int32),
        [pltpu.VMEM((CHUNK, EMB), jnp.float32) for _ in range(NBUF)],
        [pltpu.SemaphoreType.DMA for _ in range(NBUF)],
        [pltpu.SemaphoreType.DMA for _ in range(NBUF)],
    ],
)
def _gather_kernel(idx_hbm, table_hbm, out_hbm, idx_v, rows, sem_g, sem_s):
    wid = lax.axis_index("s") * NC + lax.axis_index("c")
    base = wid * PER_W

    pltpu.sync_copy(idx_hbm.at[pl.ds(base, PER_W)], idx_v)

    def fire(c, b):
        return pltpu.async_copy(
            table_hbm.at[idx_v.at[pl.ds(c * CHUNK, CHUNK)]], rows[b], sem_g[b]
        )

    def body(r, carry):
        c0 = r * NBUF
        descs = []
        for b in range(NBUF):
            @pl.when(r > 0)
            def _():
                pltpu.make_async_copy(
                    rows[b], out_hbm.at[pl.ds(0, CHUNK)], sem_s[b]
                ).wait()
            descs.append(fire(c0 + b, b))
        for b in range(NBUF):
            descs[b].wait()
            pltpu.async_copy(
                rows[b], out_hbm.at[pl.ds(base + (c0 + b) * CHUNK, CHUNK)],
                sem_s[b],
            )
        return carry

    lax.fori_loop(0, ROUNDS, body, 0)

    for b in range(NBUF):
        pltpu.make_async_copy(
            rows[b], out_hbm.at[pl.ds(0, CHUNK)], sem_s[b]
        ).wait()


def kernel(sent_words, embed_weight):
    idx = sent_words.reshape(-1).astype(jnp.int32)
    out = _gather_kernel(idx, embed_weight)
    return out.reshape(B, L, EMB)


# speedup vs baseline: 1.9319x; 1.2928x over previous
import functools

import jax
import jax.numpy as jnp
from jax import lax
from jax.experimental import pallas as pl
from jax.experimental.pallas import tpu as pltpu
from jax.experimental.pallas import tpu_sc as plsc

VOCAB = 1000000
EMB = 32
B = 4096
L = 200
N = B * L
NC = 2
NS = 16
NW = NC * NS
PER_W = N // NW
SENT_W = B // NW
NBUF = 4
ROUNDS = SENT_W // NBUF
VBLK = 8192
VGRID = -(-VOCAB // VBLK)

_mesh = plsc.VectorSubcoreMesh(core_axis_name="c", subcore_axis_name="s")


VPAD = VGRID * VBLK


def _transpose_body(wt_ref, out_ref):
    xt = jnp.transpose(wt_ref[...])
    for q in range(4):
        out_ref[:, q * EMB:(q + 1) * EMB] = xt[q * (VBLK // 4):(q + 1) * (VBLK // 4), :]


_transpose = pl.pallas_call(
    _transpose_body,
    grid=(VGRID,),
    in_specs=[pl.BlockSpec((EMB, VBLK), lambda i: (0, i))],
    out_specs=pl.BlockSpec((VBLK // 4, 128), lambda i: (i, 0)),
    out_shape=jax.ShapeDtypeStruct((VPAD * EMB // 128, 128), jnp.float32),
)


@functools.partial(
    pl.kernel,
    mesh=_mesh,
    out_type=jax.ShapeDtypeStruct((B, L, EMB), jnp.float32),
    compiler_params=pltpu.CompilerParams(use_tc_tiling_on_sc=False),
    scratch_types=[
        pltpu.VMEM((PER_W,), jnp.int32),
        [pltpu.VMEM((1, L, EMB), jnp.float32) for _ in range(NBUF)],
        [pltpu.SemaphoreType.DMA for _ in range(NBUF)],
        [pltpu.SemaphoreType.DMA for _ in range(NBUF)],
    ],
)
def _gather_kernel(idx_hbm, table_hbm, out_hbm, idx_v, rows, sem_g, sem_s):
    wid = lax.axis_index("s") * NC + lax.axis_index("c")
    base = wid * PER_W
    sbase = wid * SENT_W

    pltpu.sync_copy(idx_hbm.at[pl.ds(base, PER_W)], idx_v)

    def fire(s, b):
        return pltpu.async_copy(
            table_hbm.at[idx_v.at[pl.ds(s * L, L)]],
            rows[b].at[0],
            sem_g[b],
        )

    def store(s, b):
        return pltpu.make_async_copy(
            rows[b],
            out_hbm.at[pl.ds(sbase + s, 1)],
            sem_s[b],
        )

    def body(r, carry):
        s0 = r * NBUF
        descs = []
        for b in range(NBUF):
            @pl.when(r > 0)
            def _():
                store(0, b).wait()
            descs.append(fire(s0 + b, b))
        for b in range(NBUF):
            descs[b].wait()
            store(s0 + b, b).start()
        return carry

    lax.fori_loop(0, ROUNDS, body, 0)

    for b in range(NBUF):
        store(0, b).wait()


def kernel(sent_words, embed_weight):
    idx = sent_words.reshape(-1).astype(jnp.int32)
    rem = idx % VBLK
    idxp = (idx - rem) + 4 * (rem % (VBLK // 4)) + rem // (VBLK // 4)
    table_rm = _transpose(embed_weight.T).reshape(VPAD, EMB)
    return _gather_kernel(idxp, table_rm)
